# baseline (speedup 1.0000x reference)
import functools

import jax
import jax.numpy as jnp
from jax import lax
from jax.experimental import pallas as pl
from jax.experimental.pallas import tpu as pltpu
from jax.experimental.pallas import tpu_sc as plsc

B = 16384
NC, NS = 2, 16
NW = NC * NS
NSPLIT = 4
BS = B // NSPLIT
RPW = BS // NW
SCHUNK = 128
NSC = RPW // SCHUNK
TCH = 8
NTC = RPW // TCH

BT = 1024
D_F = 576


def _sc_gather(user_emb, zip_emb, movie_emb, title_emb, ids3, mt2):
    mesh = plsc.VectorSubcoreMesh(core_axis_name="c", subcore_axis_name="s")

    @functools.partial(
        pl.kernel,
        out_type=(
            jax.ShapeDtypeStruct((BS, 128), jnp.float32),
            jax.ShapeDtypeStruct((BS, 128), jnp.float32),
            jax.ShapeDtypeStruct((BS, 128), jnp.float32),
            jax.ShapeDtypeStruct((BS, 128), jnp.float32),
        ),
        mesh=mesh,
        scratch_types=(
            pltpu.VMEM((3 * NSC, SCHUNK), jnp.int32),
            pltpu.VMEM((NTC, TCH * 16), jnp.int32),
            pltpu.VMEM((4, SCHUNK, 128), jnp.float32),
            pltpu.VMEM((4, TCH, 128), jnp.float32),
            pltpu.SemaphoreType.DMA,
            pltpu.SemaphoreType.DMA,
            pltpu.SemaphoreType.DMA,
            pltpu.SemaphoreType.DMA,
            pltpu.SemaphoreType.DMA,
            pltpu.SemaphoreType.DMA,
            pltpu.SemaphoreType.DMA,
            pltpu.SemaphoreType.DMA,
        ),
    )
    def k(ue_t, zc_t, me_t, ti_t, ids_h, mt_h,
          ue_o, zc_o, me_o, ts_o,
          ix, tix, tvs, tsvs,
          gs0, gs1, gs2, gs3, ss0, ss1, ss2, ss3):
        wid = lax.axis_index("s") * NC + lax.axis_index("c")
        base = wid * RPW
        gsems = (gs0, gs1, gs2, gs3)
        ssems = (ss0, ss1, ss2, ss3)

        pltpu.sync_copy(ids_h.at[wid], ix)
        pltpu.sync_copy(mt_h.at[pl.ds(wid * NTC, NTC)], tix)

        plan = ([(ue_t, ue_o, c) for c in range(NSC)]
                + [(zc_t, zc_o, NSC + c) for c in range(NSC)]
                + [(me_t, me_o, 2 * NSC + c) for c in range(NSC)])
        NP = len(plan)

        def g_desc(j):
            tbl, _, r = plan[j]
            return pltpu.make_async_copy(tbl.at[ix.at[r]], tvs.at[j % 4],
                                         gsems[j % 4])

        def s_desc(j):
            _, out, r = plan[j]
            dst = out.at[pl.ds(base + (r % NSC) * SCHUNK, SCHUNK)]
            return pltpu.make_async_copy(tvs.at[j % 4], dst, ssems[j % 4])

        for j in range(min(3, NP)):
            g_desc(j).start()
        for j in range(NP):
            g_desc(j).wait()
            s_desc(j).start()
            if j + 3 < NP:
                if j - 1 >= 0:
                    s_desc(j - 1).wait()
                g_desc(j + 3).start()
        for j in range(max(NP - 4, 0), NP):
            s_desc(j).wait()

        def tg(c, k):
            return pltpu.make_async_copy(ti_t.at[tix.at[c]], tvs.at[k],
                                         gsems[k])

        def tst(c, k):
            return pltpu.make_async_copy(
                tsvs.at[k], ts_o.at[pl.ds(base + c * TCH, TCH)], ssems[k])

        def reduce16(k):
            for e in range(TCH):
                r0 = e * 16
                for f in range(8):
                    sl = pl.ds(f * 16, 16)
                    acc = tvs[k, r0, sl]
                    for j in range(1, 16):
                        acc = acc + tvs[k, r0 + j, sl]
                    tsvs[k, e, sl] = acc

        for k in range(2):
            tg(k, k).start()

        def tbody(i, carry):
            cc = 2 * i
            for k in range(2):
                c = cc + k
                tg(c, k).wait()

                @pl.when(i > 0)
                def _():
                    tst(c - 2, k).wait()

                reduce16(k)
                tst(c, k).start()

                @pl.when(i < NTC // 2 - 1)
                def _():
                    tg(c + 2, k).start()
            return carry

        lax.fori_loop(0, NTC // 2, tbody, 0)
        for k in range(2):
            tst(NTC - 2 + k, k).wait()

    return k(user_emb, zip_emb, movie_emb, title_emb, ids3, mt2)


def _tc_body(ue, zc, me, ts, scal, ints, mt, mg,
             w1, b1, s1, be1, w2, b2, s2, be2, w3, b3, s3, be3, w4, b4, t0,
             out):
    f32 = jnp.float32
    cnt_t = jnp.sum((mt[...] != 0).astype(f32), axis=1, keepdims=True)
    tavg = (ts[...] - (16.0 - cnt_t) * t0[...]) / jnp.maximum(cnt_t, 1.0)

    mgv = mg[...]
    io21 = lax.broadcasted_iota(jnp.int32, (BT, 21), 1)
    gh = (mgv[:, 0:1] == io21).astype(f32)
    for j in range(1, 6):
        gh = gh + (mgv[:, j:j + 1] == io21).astype(f32)
    gh = gh * (io21 > 0).astype(f32)
    cnt_g = jnp.sum(gh, axis=1, keepdims=True)
    ghn = gh / jnp.maximum(cnt_g, 1.0)

    iv = ints[...]

    def oh(col, K):
        io = lax.broadcasted_iota(jnp.int32, (BT, K), 1)
        return (iv[:, col:col + 1] == io).astype(f32)

    cidx = (iv[:, 3:4] * 24 + iv[:, 4:5]) % 34
    cr = (cidx == lax.broadcasted_iota(jnp.int32, (BT, 35), 1)).astype(f32)

    feats = jnp.concatenate(
        [ue[...], zc[..., 0:64], me[...], tavg, scal[...],
         oh(0, 2), oh(1, 22), oh(2, 7), oh(3, 7), oh(4, 24),
         cr, ghn, jnp.zeros((BT, 6), f32)], axis=1)

    h = jnp.dot(feats, w1[...], preferred_element_type=f32) + b1[...]
    h = jnp.maximum(h, 0.0) * s1[...] + be1[...]
    h = jnp.dot(h, w2[...], preferred_element_type=f32) + b2[...]
    h = jnp.maximum(h, 0.0) * s2[...] + be2[...]
    h = jnp.dot(h, w3[...], preferred_element_type=f32) + b3[...]
    h = jnp.maximum(h, 0.0) * s3[...] + be3[...]
    lg = jnp.dot(h, w4[...], preferred_element_type=f32) + b4[...]
    out[...] = jax.nn.sigmoid(lg)


def kernel(example_age, example_age_square, example_age_sqrt, user_emb,
           zip_emb, cross_emb, movie_emb, title_emb, genre_emb, W1, b1, g1,
           be1, W2, b2, g2, be2, W3, b3, g3, be3, W4, b4, user_gender,
           user_id, user_occupation_label, user_zip_code, bucketized_user_age,
           day_of_week, hour_of_day, movie_title, movie_id, movie_genres):
    i32 = jnp.int32
    f32 = jnp.float32

    zip_p = jnp.pad(zip_emb, ((0, 0), (0, 64)))

    bn = lax.rsqrt(jnp.float32(1.0 + 1e-3))
    s1 = (g1 * bn).reshape(1, -1)
    s2 = (g2 * bn).reshape(1, -1)
    s3 = (g3 * bn).reshape(1, -1)
    scal = jnp.stack([example_age, example_age_square, example_age_sqrt,
                      jnp.zeros((B,), f32)], axis=1)
    ints = jnp.stack([jnp.where(user_gender, 1, 0).astype(i32),
                      user_occupation_label.astype(i32),
                      bucketized_user_age.astype(i32),
                      day_of_week.astype(i32),
                      hour_of_day.astype(i32)], axis=1)
    mg = movie_genres.astype(i32)
    W1p = jnp.concatenate([
        W1[2:130], W1[152:216], W1[417:545], W1[289:417], W1[286:289],
        jnp.zeros((1, 256), f32), W1[0:2], W1[130:152], W1[216:223],
        W1[223:230], W1[230:254],
        jnp.dot(cross_emb, W1[254:286]),
        jnp.dot(genre_emb, W1[545:609]),
        jnp.zeros((6, 256), f32)], axis=0)
    t0 = title_emb[0:1, :]

    row = lambda i: (i, 0)
    full = lambda i: (0, 0)
    tc_call = pl.pallas_call(
        _tc_body,
        grid=(BS // BT,),
        in_specs=[
            pl.BlockSpec((BT, 128), row),
            pl.BlockSpec((BT, 128), row),
            pl.BlockSpec((BT, 128), row),
            pl.BlockSpec((BT, 128), row),
            pl.BlockSpec((BT, 4), row),
            pl.BlockSpec((BT, 5), row),
            pl.BlockSpec((BT, 16), row),
            pl.BlockSpec((BT, 6), row),
            pl.BlockSpec((D_F, 256), full),
            pl.BlockSpec((1, 256), full),
            pl.BlockSpec((1, 256), full),
            pl.BlockSpec((1, 256), full),
            pl.BlockSpec((256, 128), full),
            pl.BlockSpec((1, 128), full),
            pl.BlockSpec((1, 128), full),
            pl.BlockSpec((1, 128), full),
            pl.BlockSpec((128, 64), full),
            pl.BlockSpec((1, 64), full),
            pl.BlockSpec((1, 64), full),
            pl.BlockSpec((1, 64), full),
            pl.BlockSpec((64, 1), full),
            pl.BlockSpec((1, 1), full),
            pl.BlockSpec((1, 128), full),
        ],
        out_specs=pl.BlockSpec((BT, 1), row),
        out_shape=jax.ShapeDtypeStruct((BS, 1), f32),
    )

    uid_i = user_id.astype(i32)
    zid_i = user_zip_code.astype(i32)
    mid_i = movie_id.astype(i32)
    mt_i = movie_title.astype(i32)
    outs = []
    for h in range(NSPLIT):
        sl = slice(h * BS, (h + 1) * BS)
        ids3 = jnp.concatenate(
            [uid_i[sl].reshape(NW, NSC, SCHUNK),
             zid_i[sl].reshape(NW, NSC, SCHUNK),
             mid_i[sl].reshape(NW, NSC, SCHUNK)], axis=1)
        mt2 = mt_i[sl].reshape(NW * NTC, TCH * 16)
        ue, zc, me, ts = _sc_gather(user_emb, zip_p, movie_emb, title_emb,
                                    ids3, mt2)
        outs.append(tc_call(
            ue, zc, me, ts, scal[sl], ints[sl], mt_i[sl], mg[sl],
            W1p, b1.reshape(1, -1), s1, be1.reshape(1, -1),
            W2, b2.reshape(1, -1), s2, be2.reshape(1, -1),
            W3, b3.reshape(1, -1), s3, be3.reshape(1, -1),
            W4, b4.reshape(1, -1), t0))
    return jnp.concatenate(outs, axis=0)


# speedup vs baseline: 1.0688x; 1.0688x over previous
import functools

import jax
import jax.numpy as jnp
from jax import lax
from jax.experimental import pallas as pl
from jax.experimental.pallas import tpu as pltpu
from jax.experimental.pallas import tpu_sc as plsc

B = 16384
NC, NS = 2, 16
NW = NC * NS
NSPLIT = 2
BS = B // NSPLIT
RPW = BS // NW
SCHUNK = 128
NSC = RPW // SCHUNK
TCH = 8
NTC = RPW // TCH

BT = 1024
D_F = 576


def _sc_gather(user_emb, zip_emb, movie_emb, title_emb, ids3, mt2):
    mesh = plsc.VectorSubcoreMesh(core_axis_name="c", subcore_axis_name="s")

    @functools.partial(
        pl.kernel,
        out_type=(
            jax.ShapeDtypeStruct((BS, 128), jnp.float32),
            jax.ShapeDtypeStruct((BS, 128), jnp.float32),
            jax.ShapeDtypeStruct((BS, 128), jnp.float32),
            jax.ShapeDtypeStruct((BS, 128), jnp.float32),
        ),
        mesh=mesh,
        scratch_types=(
            pltpu.VMEM((3 * NSC, SCHUNK), jnp.int32),
            pltpu.VMEM((NTC, TCH * 16), jnp.int32),
            pltpu.VMEM((4, SCHUNK, 128), jnp.float32),
            pltpu.VMEM((4, TCH, 128), jnp.float32),
            pltpu.SemaphoreType.DMA,
            pltpu.SemaphoreType.DMA,
            pltpu.SemaphoreType.DMA,
            pltpu.SemaphoreType.DMA,
            pltpu.SemaphoreType.DMA,
            pltpu.SemaphoreType.DMA,
            pltpu.SemaphoreType.DMA,
            pltpu.SemaphoreType.DMA,
        ),
    )
    def k(ue_t, zc_t, me_t, ti_t, ids_h, mt_h,
          ue_o, zc_o, me_o, ts_o,
          ix, tix, tvs, tsvs,
          gs0, gs1, gs2, gs3, ss0, ss1, ss2, ss3):
        wid = lax.axis_index("s") * NC + lax.axis_index("c")
        base = wid * RPW
        gsems = (gs0, gs1, gs2, gs3)
        ssems = (ss0, ss1, ss2, ss3)

        pltpu.sync_copy(ids_h.at[wid], ix)
        pltpu.sync_copy(mt_h.at[pl.ds(wid * NTC, NTC)], tix)

        plan = ([(ue_t, ue_o, c) for c in range(NSC)]
                + [(zc_t, zc_o, NSC + c) for c in range(NSC)]
                + [(me_t, me_o, 2 * NSC + c) for c in range(NSC)])
        NP = len(plan)

        def g_desc(j):
            tbl, _, r = plan[j]
            return pltpu.make_async_copy(tbl.at[ix.at[r]], tvs.at[j % 4],
                                         gsems[j % 4])

        def s_desc(j):
            _, out, r = plan[j]
            dst = out.at[pl.ds(base + (r % NSC) * SCHUNK, SCHUNK)]
            return pltpu.make_async_copy(tvs.at[j % 4], dst, ssems[j % 4])

        for j in range(min(3, NP)):
            g_desc(j).start()
        for j in range(NP):
            g_desc(j).wait()
            s_desc(j).start()
            if j + 3 < NP:
                if j - 1 >= 0:
                    s_desc(j - 1).wait()
                g_desc(j + 3).start()
        for j in range(max(NP - 4, 0), NP):
            s_desc(j).wait()

        def tg(c, k):
            return pltpu.make_async_copy(ti_t.at[tix.at[c]], tvs.at[k],
                                         gsems[k])

        def tst(c, k):
            return pltpu.make_async_copy(
                tsvs.at[k], ts_o.at[pl.ds(base + c * TCH, TCH)], ssems[k])

        def reduce16(k):
            for e in range(TCH):
                r0 = e * 16
                for f in range(8):
                    sl = pl.ds(f * 16, 16)
                    acc = tvs[k, r0, sl]
                    for j in range(1, 16):
                        acc = acc + tvs[k, r0 + j, sl]
                    tsvs[k, e, sl] = acc

        for k in range(2):
            tg(k, k).start()

        def tbody(i, carry):
            cc = 2 * i
            for k in range(2):
                c = cc + k
                tg(c, k).wait()

                @pl.when(i > 0)
                def _():
                    tst(c - 2, k).wait()

                reduce16(k)
                tst(c, k).start()

                @pl.when(i < NTC // 2 - 1)
                def _():
                    tg(c + 2, k).start()
            return carry

        lax.fori_loop(0, NTC // 2, tbody, 0)
        for k in range(2):
            tst(NTC - 2 + k, k).wait()

    return k(user_emb, zip_emb, movie_emb, title_emb, ids3, mt2)


def _tc_body(ue, zc, me, ts, scal, ints, mt, mg,
             w1, b1, s1, be1, w2, b2, s2, be2, w3, b3, s3, be3, w4, b4, t0,
             out):
    f32 = jnp.float32
    cnt_t = jnp.sum((mt[...] != 0).astype(f32), axis=1, keepdims=True)
    tavg = (ts[...] - (16.0 - cnt_t) * t0[...]) / jnp.maximum(cnt_t, 1.0)

    mgv = mg[...]
    io21 = lax.broadcasted_iota(jnp.int32, (BT, 21), 1)
    gh = (mgv[:, 0:1] == io21).astype(f32)
    for j in range(1, 6):
        gh = gh + (mgv[:, j:j + 1] == io21).astype(f32)
    gh = gh * (io21 > 0).astype(f32)
    cnt_g = jnp.sum(gh, axis=1, keepdims=True)
    ghn = gh / jnp.maximum(cnt_g, 1.0)

    iv = ints[...]

    def oh(col, K):
        io = lax.broadcasted_iota(jnp.int32, (BT, K), 1)
        return (iv[:, col:col + 1] == io).astype(f32)

    cidx = (iv[:, 3:4] * 24 + iv[:, 4:5]) % 34
    cr = (cidx == lax.broadcasted_iota(jnp.int32, (BT, 35), 1)).astype(f32)

    feats = jnp.concatenate(
        [ue[...], zc[..., 0:64], me[...], tavg, scal[...],
         oh(0, 2), oh(1, 22), oh(2, 7), oh(3, 7), oh(4, 24),
         cr, ghn, jnp.zeros((BT, 6), f32)], axis=1)

    h = jnp.dot(feats, w1[...], preferred_element_type=f32) + b1[...]
    h = jnp.maximum(h, 0.0) * s1[...] + be1[...]
    h = jnp.dot(h, w2[...], preferred_element_type=f32) + b2[...]
    h = jnp.maximum(h, 0.0) * s2[...] + be2[...]
    h = jnp.dot(h, w3[...], preferred_element_type=f32) + b3[...]
    h = jnp.maximum(h, 0.0) * s3[...] + be3[...]
    lg = jnp.dot(h, w4[...], preferred_element_type=f32) + b4[...]
    out[...] = jax.nn.sigmoid(lg)


def kernel(example_age, example_age_square, example_age_sqrt, user_emb,
           zip_emb, cross_emb, movie_emb, title_emb, genre_emb, W1, b1, g1,
           be1, W2, b2, g2, be2, W3, b3, g3, be3, W4, b4, user_gender,
           user_id, user_occupation_label, user_zip_code, bucketized_user_age,
           day_of_week, hour_of_day, movie_title, movie_id, movie_genres):
    i32 = jnp.int32
    f32 = jnp.float32

    zip_p = jnp.pad(zip_emb, ((0, 0), (0, 64)))

    bn = lax.rsqrt(jnp.float32(1.0 + 1e-3))
    s1 = (g1 * bn).reshape(1, -1)
    s2 = (g2 * bn).reshape(1, -1)
    s3 = (g3 * bn).reshape(1, -1)
    scal = jnp.stack([example_age, example_age_square, example_age_sqrt,
                      jnp.zeros((B,), f32)], axis=1)
    ints = jnp.stack([jnp.where(user_gender, 1, 0).astype(i32),
                      user_occupation_label.astype(i32),
                      bucketized_user_age.astype(i32),
                      day_of_week.astype(i32),
                      hour_of_day.astype(i32)], axis=1)
    mg = movie_genres.astype(i32)
    W1p = jnp.concatenate([
        W1[2:130], W1[152:216], W1[417:545], W1[289:417], W1[286:289],
        jnp.zeros((1, 256), f32), W1[0:2], W1[130:152], W1[216:223],
        W1[223:230], W1[230:254],
        jnp.dot(cross_emb, W1[254:286]),
        jnp.dot(genre_emb, W1[545:609]),
        jnp.zeros((6, 256), f32)], axis=0)
    t0 = title_emb[0:1, :]

    row = lambda i: (i, 0)
    full = lambda i: (0, 0)
    tc_call = pl.pallas_call(
        _tc_body,
        grid=(BS // BT,),
        in_specs=[
            pl.BlockSpec((BT, 128), row),
            pl.BlockSpec((BT, 128), row),
            pl.BlockSpec((BT, 128), row),
            pl.BlockSpec((BT, 128), row),
            pl.BlockSpec((BT, 4), row),
            pl.BlockSpec((BT, 5), row),
            pl.BlockSpec((BT, 16), row),
            pl.BlockSpec((BT, 6), row),
            pl.BlockSpec((D_F, 256), full),
            pl.BlockSpec((1, 256), full),
            pl.BlockSpec((1, 256), full),
            pl.BlockSpec((1, 256), full),
            pl.BlockSpec((256, 128), full),
            pl.BlockSpec((1, 128), full),
            pl.BlockSpec((1, 128), full),
            pl.BlockSpec((1, 128), full),
            pl.BlockSpec((128, 64), full),
            pl.BlockSpec((1, 64), full),
            pl.BlockSpec((1, 64), full),
            pl.BlockSpec((1, 64), full),
            pl.BlockSpec((64, 1), full),
            pl.BlockSpec((1, 1), full),
            pl.BlockSpec((1, 128), full),
        ],
        out_specs=pl.BlockSpec((BT, 1), row),
        out_shape=jax.ShapeDtypeStruct((BS, 1), f32),
    )

    uid_i = user_id.astype(i32)
    zid_i = user_zip_code.astype(i32)
    mid_i = movie_id.astype(i32)
    mt_i = movie_title.astype(i32)
    outs = []
    for h in range(NSPLIT):
        sl = slice(h * BS, (h + 1) * BS)
        ids3 = jnp.concatenate(
            [uid_i[sl].reshape(NW, NSC, SCHUNK),
             zid_i[sl].reshape(NW, NSC, SCHUNK),
             mid_i[sl].reshape(NW, NSC, SCHUNK)], axis=1)
        mt2 = mt_i[sl].reshape(NW * NTC, TCH * 16)
        ue, zc, me, ts = _sc_gather(user_emb, zip_p, movie_emb, title_emb,
                                    ids3, mt2)
        outs.append(tc_call(
            ue, zc, me, ts, scal[sl], ints[sl], mt_i[sl], mg[sl],
            W1p, b1.reshape(1, -1), s1, be1.reshape(1, -1),
            W2, b2.reshape(1, -1), s2, be2.reshape(1, -1),
            W3, b3.reshape(1, -1), s3, be3.reshape(1, -1),
            W4, b4.reshape(1, -1), t0))
    return jnp.concatenate(outs, axis=0)


# speedup vs baseline: 1.3298x; 1.2442x over previous
import functools

import jax
import jax.numpy as jnp
from jax import lax
from jax.experimental import pallas as pl
from jax.experimental.pallas import tpu as pltpu
from jax.experimental.pallas import tpu_sc as plsc

B = 16384
NC, NS = 2, 16
NW = NC * NS
NSPLIT = 2
BS = B // NSPLIT
RPW = BS // NW
SCHUNK = 128
NSC = RPW // SCHUNK
TCH = 8
NTC = RPW // TCH

BT = 1024
D_F = 576


def _sc_gather(user_emb, zip_emb, movie_emb, title_emb, ids3, mt2):
    mesh = plsc.VectorSubcoreMesh(core_axis_name="c", subcore_axis_name="s")

    @functools.partial(
        pl.kernel,
        out_type=(
            jax.ShapeDtypeStruct((BS, 128), jnp.float32),
            jax.ShapeDtypeStruct((BS, 128), jnp.float32),
            jax.ShapeDtypeStruct((BS, 128), jnp.float32),
            jax.ShapeDtypeStruct((BS, 128), jnp.float32),
        ),
        mesh=mesh,
        scratch_types=(
            pltpu.VMEM((3 * NSC, SCHUNK), jnp.int32),
            pltpu.VMEM((NTC, TCH * 16), jnp.int32),
            pltpu.VMEM((4, SCHUNK, 128), jnp.float32),
            pltpu.VMEM((4, TCH, 128), jnp.float32),
            pltpu.SemaphoreType.DMA,
            pltpu.SemaphoreType.DMA,
            pltpu.SemaphoreType.DMA,
            pltpu.SemaphoreType.DMA,
            pltpu.SemaphoreType.DMA,
            pltpu.SemaphoreType.DMA,
            pltpu.SemaphoreType.DMA,
            pltpu.SemaphoreType.DMA,
        ),
    )
    def k(ue_t, zc_t, me_t, ti_t, ids_h, mt_h,
          ue_o, zc_o, me_o, ts_o,
          ix, tix, tvs, tsvs,
          gs0, gs1, gs2, gs3, ss0, ss1, ss2, ss3):
        wid = lax.axis_index("s") * NC + lax.axis_index("c")
        base = wid * RPW
        gsems = (gs0, gs1, gs2, gs3)
        ssems = (ss0, ss1, ss2, ss3)

        pltpu.sync_copy(ids_h.at[wid], ix)
        pltpu.sync_copy(mt_h.at[pl.ds(wid * NTC, NTC)], tix)

        plan = ([(ue_t, ue_o, c) for c in range(NSC)]
                + [(zc_t, zc_o, NSC + c) for c in range(NSC)]
                + [(me_t, me_o, 2 * NSC + c) for c in range(NSC)])
        NP = len(plan)

        def g_desc(j):
            tbl, _, r = plan[j]
            return pltpu.make_async_copy(tbl.at[ix.at[r]], tvs.at[j % 4],
                                         gsems[j % 4])

        def s_desc(j):
            _, out, r = plan[j]
            dst = out.at[pl.ds(base + (r % NSC) * SCHUNK, SCHUNK)]
            return pltpu.make_async_copy(tvs.at[j % 4], dst, ssems[j % 4])

        for j in range(min(3, NP)):
            g_desc(j).start()
        for j in range(NP):
            g_desc(j).wait()
            s_desc(j).start()
            if j + 3 < NP:
                if j - 1 >= 0:
                    s_desc(j - 1).wait()
                g_desc(j + 3).start()
        for j in range(max(NP - 4, 0), NP):
            s_desc(j).wait()

        def tg(c, k):
            return pltpu.make_async_copy(ti_t.at[tix.at[c]], tvs.at[k],
                                         gsems[k])

        def tst(c, k):
            return pltpu.make_async_copy(
                tsvs.at[k], ts_o.at[pl.ds(base + c * TCH, TCH)], ssems[k])

        def reduce16(k):
            def ebody(e, carry):
                r0 = e * 16
                for f in range(8):
                    sl = pl.ds(f * 16, 16)
                    acc = tvs[k, r0, sl]
                    for j in range(1, 16):
                        acc = acc + tvs[k, r0 + j, sl]
                    tsvs[k, e, sl] = acc
                return carry
            lax.fori_loop(0, TCH, ebody, 0)

        for k in range(4):
            tg(k, k).start()

        def tbody(i, carry):
            cc = 4 * i
            for k in range(4):
                c = cc + k
                tg(c, k).wait()

                @pl.when(i > 0)
                def _():
                    tst(c - 4, k).wait()

                reduce16(k)
                tst(c, k).start()

                @pl.when(i < NTC // 4 - 1)
                def _():
                    tg(c + 4, k).start()
            return carry

        lax.fori_loop(0, NTC // 4, tbody, 0)
        for k in range(4):
            tst(NTC - 4 + k, k).wait()

    return k(user_emb, zip_emb, movie_emb, title_emb, ids3, mt2)


def _tc_body(ue, zc, me, ts, scal, ints, mt, mg,
             w1, b1, s1, be1, w2, b2, s2, be2, w3, b3, s3, be3, w4, b4, t0,
             out):
    f32 = jnp.float32
    cnt_t = jnp.sum((mt[...] != 0).astype(f32), axis=1, keepdims=True)
    tavg = (ts[...] - (16.0 - cnt_t) * t0[...]) / jnp.maximum(cnt_t, 1.0)

    mgv = mg[...]
    io21 = lax.broadcasted_iota(jnp.int32, (BT, 21), 1)
    gh = (mgv[:, 0:1] == io21).astype(f32)
    for j in range(1, 6):
        gh = gh + (mgv[:, j:j + 1] == io21).astype(f32)
    gh = gh * (io21 > 0).astype(f32)
    cnt_g = jnp.sum(gh, axis=1, keepdims=True)
    ghn = gh / jnp.maximum(cnt_g, 1.0)

    iv = ints[...]

    def oh(col, K):
        io = lax.broadcasted_iota(jnp.int32, (BT, K), 1)
        return (iv[:, col:col + 1] == io).astype(f32)

    cidx = (iv[:, 3:4] * 24 + iv[:, 4:5]) % 34
    cr = (cidx == lax.broadcasted_iota(jnp.int32, (BT, 35), 1)).astype(f32)

    feats = jnp.concatenate(
        [ue[...], zc[..., 0:64], me[...], tavg, scal[...],
         oh(0, 2), oh(1, 22), oh(2, 7), oh(3, 7), oh(4, 24),
         cr, ghn, jnp.zeros((BT, 6), f32)], axis=1)

    h = jnp.dot(feats, w1[...], preferred_element_type=f32) + b1[...]
    h = jnp.maximum(h, 0.0) * s1[...] + be1[...]
    h = jnp.dot(h, w2[...], preferred_element_type=f32) + b2[...]
    h = jnp.maximum(h, 0.0) * s2[...] + be2[...]
    h = jnp.dot(h, w3[...], preferred_element_type=f32) + b3[...]
    h = jnp.maximum(h, 0.0) * s3[...] + be3[...]
    lg = jnp.dot(h, w4[...], preferred_element_type=f32) + b4[...]
    out[...] = jax.nn.sigmoid(lg)


def kernel(example_age, example_age_square, example_age_sqrt, user_emb,
           zip_emb, cross_emb, movie_emb, title_emb, genre_emb, W1, b1, g1,
           be1, W2, b2, g2, be2, W3, b3, g3, be3, W4, b4, user_gender,
           user_id, user_occupation_label, user_zip_code, bucketized_user_age,
           day_of_week, hour_of_day, movie_title, movie_id, movie_genres):
    i32 = jnp.int32
    f32 = jnp.float32

    zip_p = jnp.pad(zip_emb, ((0, 0), (0, 64)))

    bn = lax.rsqrt(jnp.float32(1.0 + 1e-3))
    s1 = (g1 * bn).reshape(1, -1)
    s2 = (g2 * bn).reshape(1, -1)
    s3 = (g3 * bn).reshape(1, -1)
    scal = jnp.stack([example_age, example_age_square, example_age_sqrt,
                      jnp.zeros((B,), f32)], axis=1)
    ints = jnp.stack([jnp.where(user_gender, 1, 0).astype(i32),
                      user_occupation_label.astype(i32),
                      bucketized_user_age.astype(i32),
                      day_of_week.astype(i32),
                      hour_of_day.astype(i32)], axis=1)
    mg = movie_genres.astype(i32)
    W1p = jnp.concatenate([
        W1[2:130], W1[152:216], W1[417:545], W1[289:417], W1[286:289],
        jnp.zeros((1, 256), f32), W1[0:2], W1[130:152], W1[216:223],
        W1[223:230], W1[230:254],
        jnp.dot(cross_emb, W1[254:286]),
        jnp.dot(genre_emb, W1[545:609]),
        jnp.zeros((6, 256), f32)], axis=0)
    t0 = title_emb[0:1, :]

    row = lambda i: (i, 0)
    full = lambda i: (0, 0)
    tc_call = pl.pallas_call(
        _tc_body,
        grid=(BS // BT,),
        in_specs=[
            pl.BlockSpec((BT, 128), row),
            pl.BlockSpec((BT, 128), row),
            pl.BlockSpec((BT, 128), row),
            pl.BlockSpec((BT, 128), row),
            pl.BlockSpec((BT, 4), row),
            pl.BlockSpec((BT, 5), row),
            pl.BlockSpec((BT, 16), row),
            pl.BlockSpec((BT, 6), row),
            pl.BlockSpec((D_F, 256), full),
            pl.BlockSpec((1, 256), full),
            pl.BlockSpec((1, 256), full),
            pl.BlockSpec((1, 256), full),
            pl.BlockSpec((256, 128), full),
            pl.BlockSpec((1, 128), full),
            pl.BlockSpec((1, 128), full),
            pl.BlockSpec((1, 128), full),
            pl.BlockSpec((128, 64), full),
            pl.BlockSpec((1, 64), full),
            pl.BlockSpec((1, 64), full),
            pl.BlockSpec((1, 64), full),
            pl.BlockSpec((64, 1), full),
            pl.BlockSpec((1, 1), full),
            pl.BlockSpec((1, 128), full),
        ],
        out_specs=pl.BlockSpec((BT, 1), row),
        out_shape=jax.ShapeDtypeStruct((BS, 1), f32),
    )

    uid_i = user_id.astype(i32)
    zid_i = user_zip_code.astype(i32)
    mid_i = movie_id.astype(i32)
    mt_i = movie_title.astype(i32)
    outs = []
    for h in range(NSPLIT):
        sl = slice(h * BS, (h + 1) * BS)
        ids3 = jnp.concatenate(
            [uid_i[sl].reshape(NW, NSC, SCHUNK),
             zid_i[sl].reshape(NW, NSC, SCHUNK),
             mid_i[sl].reshape(NW, NSC, SCHUNK)], axis=1)
        mt2 = mt_i[sl].reshape(NW * NTC, TCH * 16)
        ue, zc, me, ts = _sc_gather(user_emb, zip_p, movie_emb, title_emb,
                                    ids3, mt2)
        outs.append(tc_call(
            ue, zc, me, ts, scal[sl], ints[sl], mt_i[sl], mg[sl],
            W1p, b1.reshape(1, -1), s1, be1.reshape(1, -1),
            W2, b2.reshape(1, -1), s2, be2.reshape(1, -1),
            W3, b3.reshape(1, -1), s3, be3.reshape(1, -1),
            W4, b4.reshape(1, -1), t0))
    return jnp.concatenate(outs, axis=0)


# speedup vs baseline: 1.3455x; 1.0118x over previous
import functools

import jax
import jax.numpy as jnp
from jax import lax
from jax.experimental import pallas as pl
from jax.experimental.pallas import tpu as pltpu
from jax.experimental.pallas import tpu_sc as plsc

B = 16384
NC, NS = 2, 16
NW = NC * NS
NSPLIT = 2
BS = B // NSPLIT
RPW = BS // NW
SCHUNK = 128
NSC = RPW // SCHUNK
TCH = 8
NTC = RPW // TCH

BT = 1024
D_F = 576


def _sc_gather(user_emb, zip_emb, movie_emb, title_emb, ids3, mt2):
    mesh = plsc.VectorSubcoreMesh(core_axis_name="c", subcore_axis_name="s")

    @functools.partial(
        pl.kernel,
        out_type=(
            jax.ShapeDtypeStruct((BS, 128), jnp.float32),
            jax.ShapeDtypeStruct((BS, 128), jnp.float32),
            jax.ShapeDtypeStruct((BS, 128), jnp.float32),
            jax.ShapeDtypeStruct((BS, 128), jnp.float32),
        ),
        mesh=mesh,
        scratch_types=(
            pltpu.VMEM((3 * NSC, SCHUNK), jnp.int32),
            pltpu.VMEM((NTC, TCH * 16), jnp.int32),
            pltpu.VMEM((4, SCHUNK, 128), jnp.float32),
            pltpu.VMEM((4, TCH, 128), jnp.float32),
            pltpu.SemaphoreType.DMA,
            pltpu.SemaphoreType.DMA,
            pltpu.SemaphoreType.DMA,
            pltpu.SemaphoreType.DMA,
            pltpu.SemaphoreType.DMA,
            pltpu.SemaphoreType.DMA,
            pltpu.SemaphoreType.DMA,
            pltpu.SemaphoreType.DMA,
        ),
    )
    def k(ue_t, zc_t, me_t, ti_t, ids_h, mt_h,
          ue_o, zc_o, me_o, ts_o,
          ix, tix, tvs, tsvs,
          gs0, gs1, gs2, gs3, ss0, ss1, ss2, ss3):
        wid = lax.axis_index("s") * NC + lax.axis_index("c")
        base = wid * RPW
        gsems = (gs0, gs1, gs2, gs3)
        ssems = (ss0, ss1, ss2, ss3)

        pltpu.sync_copy(ids_h.at[wid], ix)
        pltpu.sync_copy(mt_h.at[pl.ds(wid * NTC, NTC)], tix)

        plan = ([(ue_t, ue_o, c) for c in range(NSC)]
                + [(zc_t, zc_o, NSC + c) for c in range(NSC)]
                + [(me_t, me_o, 2 * NSC + c) for c in range(NSC)])
        NP = len(plan)

        def g_desc(j):
            tbl, _, r = plan[j]
            return pltpu.make_async_copy(tbl.at[ix.at[r]], tvs.at[j % 4],
                                         gsems[j % 4])

        def s_desc(j):
            _, out, r = plan[j]
            dst = out.at[pl.ds(base + (r % NSC) * SCHUNK, SCHUNK)]
            return pltpu.make_async_copy(tvs.at[j % 4], dst, ssems[j % 4])

        for j in range(min(3, NP)):
            g_desc(j).start()
        for j in range(NP):
            g_desc(j).wait()
            s_desc(j).start()
            if j + 3 < NP:
                if j - 1 >= 0:
                    s_desc(j - 1).wait()
                g_desc(j + 3).start()
        for j in range(max(NP - 4, 0), NP):
            s_desc(j).wait()

        def tg(c, k):
            return pltpu.make_async_copy(ti_t.at[tix.at[c]], tvs.at[k],
                                         gsems[k])

        def tst(c, k):
            return pltpu.make_async_copy(
                tsvs.at[k], ts_o.at[pl.ds(base + c * TCH, TCH)], ssems[k])

        def reduce16(k):
            def ebody(e, carry):
                r0 = e * 16
                for f in range(8):
                    sl = pl.ds(f * 16, 16)
                    acc = tvs[k, r0, sl]
                    for j in range(1, 16):
                        acc = acc + tvs[k, r0 + j, sl]
                    tsvs[k, e, sl] = acc
                return carry
            lax.fori_loop(0, TCH, ebody, 0)

        for k in range(4):
            tg(k, k).start()

        def tbody(i, carry):
            cc = 4 * i
            for k in range(4):
                c = cc + k
                tg(c, k).wait()

                @pl.when(i > 0)
                def _():
                    tst(c - 4, k).wait()

                reduce16(k)
                tst(c, k).start()

                @pl.when(i < NTC // 4 - 1)
                def _():
                    tg(c + 4, k).start()
            return carry

        lax.fori_loop(0, NTC // 4, tbody, 0)
        for k in range(4):
            tst(NTC - 4 + k, k).wait()

    return k(user_emb, zip_emb, movie_emb, title_emb, ids3, mt2)


def _tc_body(ue, zc, me, ts, scal, ints, mt, mg,
             w1u, w1z, w1m, w1t, w1s, b1, s1, be1,
             w2, b2, s2, be2, w3, b3, s3, be3, w4, b4, t0,
             out):
    f32 = jnp.float32
    cnt_t = jnp.sum((mt[...] != 0).astype(f32), axis=1, keepdims=True)
    tavg = (ts[...] - (16.0 - cnt_t) * t0[...]) / jnp.maximum(cnt_t, 1.0)

    mgv = mg[...]
    io21 = lax.broadcasted_iota(jnp.int32, (BT, 21), 1)
    gh = (mgv[:, 0:1] == io21).astype(f32)
    for j in range(1, 6):
        gh = gh + (mgv[:, j:j + 1] == io21).astype(f32)
    gh = gh * (io21 > 0).astype(f32)
    cnt_g = jnp.sum(gh, axis=1, keepdims=True)
    ghn = gh / jnp.maximum(cnt_g, 1.0)

    iv = ints[...]

    def oh(col, K):
        io = lax.broadcasted_iota(jnp.int32, (BT, K), 1)
        return (iv[:, col:col + 1] == io).astype(f32)

    cidx = (iv[:, 3:4] * 24 + iv[:, 4:5]) % 34
    cr = (cidx == lax.broadcasted_iota(jnp.int32, (BT, 35), 1)).astype(f32)

    small = jnp.concatenate(
        [scal[...], oh(0, 2), oh(1, 22), oh(2, 7), oh(3, 7), oh(4, 24),
         cr, ghn, jnp.zeros((BT, 6), f32)], axis=1)

    h = (jnp.dot(ue[...], w1u[...], preferred_element_type=f32)
         + jnp.dot(zc[...], w1z[...], preferred_element_type=f32)
         + jnp.dot(me[...], w1m[...], preferred_element_type=f32)
         + jnp.dot(tavg, w1t[...], preferred_element_type=f32)
         + jnp.dot(small, w1s[...], preferred_element_type=f32)
         + b1[...])
    h = jnp.maximum(h, 0.0) * s1[...] + be1[...]
    h = jnp.dot(h, w2[...], preferred_element_type=f32) + b2[...]
    h = jnp.maximum(h, 0.0) * s2[...] + be2[...]
    h = jnp.dot(h, w3[...], preferred_element_type=f32) + b3[...]
    h = jnp.maximum(h, 0.0) * s3[...] + be3[...]
    lg = jnp.dot(h, w4[...], preferred_element_type=f32) + b4[...]
    out[...] = jax.nn.sigmoid(lg)


def kernel(example_age, example_age_square, example_age_sqrt, user_emb,
           zip_emb, cross_emb, movie_emb, title_emb, genre_emb, W1, b1, g1,
           be1, W2, b2, g2, be2, W3, b3, g3, be3, W4, b4, user_gender,
           user_id, user_occupation_label, user_zip_code, bucketized_user_age,
           day_of_week, hour_of_day, movie_title, movie_id, movie_genres):
    i32 = jnp.int32
    f32 = jnp.float32

    zip_p = jnp.pad(zip_emb, ((0, 0), (0, 64)))

    bn = lax.rsqrt(jnp.float32(1.0 + 1e-3))
    s1 = (g1 * bn).reshape(1, -1)
    s2 = (g2 * bn).reshape(1, -1)
    s3 = (g3 * bn).reshape(1, -1)
    scal = jnp.stack([example_age, example_age_square, example_age_sqrt,
                      jnp.zeros((B,), f32)], axis=1)
    ints = jnp.stack([jnp.where(user_gender, 1, 0).astype(i32),
                      user_occupation_label.astype(i32),
                      bucketized_user_age.astype(i32),
                      day_of_week.astype(i32),
                      hour_of_day.astype(i32)], axis=1)
    mg = movie_genres.astype(i32)
    W1u = W1[2:130]
    W1z = jnp.concatenate([W1[152:216], jnp.zeros((64, 256), f32)], axis=0)
    W1m = W1[417:545]
    W1t = W1[289:417]
    W1s = jnp.concatenate([
        W1[286:289], jnp.zeros((1, 256), f32), W1[0:2], W1[130:152],
        W1[216:223], W1[223:230], W1[230:254],
        jnp.dot(cross_emb, W1[254:286]),
        jnp.dot(genre_emb, W1[545:609]),
        jnp.zeros((6, 256), f32)], axis=0)
    t0 = title_emb[0:1, :]

    row = lambda i: (i, 0)
    full = lambda i: (0, 0)
    tc_call = pl.pallas_call(
        _tc_body,
        grid=(BS // BT,),
        in_specs=[
            pl.BlockSpec((BT, 128), row),
            pl.BlockSpec((BT, 128), row),
            pl.BlockSpec((BT, 128), row),
            pl.BlockSpec((BT, 128), row),
            pl.BlockSpec((BT, 4), row),
            pl.BlockSpec((BT, 5), row),
            pl.BlockSpec((BT, 16), row),
            pl.BlockSpec((BT, 6), row),
            pl.BlockSpec((128, 256), full),
            pl.BlockSpec((128, 256), full),
            pl.BlockSpec((128, 256), full),
            pl.BlockSpec((128, 256), full),
            pl.BlockSpec((128, 256), full),
            pl.BlockSpec((1, 256), full),
            pl.BlockSpec((1, 256), full),
            pl.BlockSpec((1, 256), full),
            pl.BlockSpec((256, 128), full),
            pl.BlockSpec((1, 128), full),
            pl.BlockSpec((1, 128), full),
            pl.BlockSpec((1, 128), full),
            pl.BlockSpec((128, 64), full),
            pl.BlockSpec((1, 64), full),
            pl.BlockSpec((1, 64), full),
            pl.BlockSpec((1, 64), full),
            pl.BlockSpec((64, 1), full),
            pl.BlockSpec((1, 1), full),
            pl.BlockSpec((1, 128), full),
        ],
        out_specs=pl.BlockSpec((BT, 1), row),
        out_shape=jax.ShapeDtypeStruct((BS, 1), f32),
    )

    uid_i = user_id.astype(i32)
    zid_i = user_zip_code.astype(i32)
    mid_i = movie_id.astype(i32)
    mt_i = movie_title.astype(i32)
    outs = []
    for h in range(NSPLIT):
        sl = slice(h * BS, (h + 1) * BS)
        ids3 = jnp.concatenate(
            [uid_i[sl].reshape(NW, NSC, SCHUNK),
             zid_i[sl].reshape(NW, NSC, SCHUNK),
             mid_i[sl].reshape(NW, NSC, SCHUNK)], axis=1)
        mt2 = mt_i[sl].reshape(NW * NTC, TCH * 16)
        ue, zc, me, ts = _sc_gather(user_emb, zip_p, movie_emb, title_emb,
                                    ids3, mt2)
        outs.append(tc_call(
            ue, zc, me, ts, scal[sl], ints[sl], mt_i[sl], mg[sl],
            W1u, W1z, W1m, W1t, W1s, b1.reshape(1, -1), s1, be1.reshape(1, -1),
            W2, b2.reshape(1, -1), s2, be2.reshape(1, -1),
            W3, b3.reshape(1, -1), s3, be3.reshape(1, -1),
            W4, b4.reshape(1, -1), t0))
    return jnp.concatenate(outs, axis=0)
